# baseline (speedup 1.0000x reference)
import functools
import math

import jax
import jax.numpy as jnp
from jax.experimental import pallas as pl
from jax.experimental.pallas import tpu as pltpu

_R_BLK = 256


def _body(months_ref, ce_ref, pe_ref, mt_ref, ratio_ref, tok_ref, out_ref,
          *, t, s, w, d4, r_blk):
    half = d4 // 4

    pe = pe_ref[:t, :]
    m = months_ref[0]
    oh = (m == jax.lax.broadcasted_iota(jnp.int32, (t, 12), 1)).astype(jnp.float32)
    me = jnp.dot(oh, mt_ref[:, :], preferred_element_type=jnp.float32)
    zt = jnp.zeros((t, d4), dtype=jnp.float32)
    a_t = jnp.concatenate([zt, pe, me, zt], axis=-1)

    r0 = (pl.program_id(1) * r_blk
          + jax.lax.broadcasted_iota(jnp.int32, (r_blk, 1), 0))
    s_idx = jax.lax.rem(r0, s)
    oh_s = (s_idx == jax.lax.broadcasted_iota(jnp.int32, (r_blk, s), 1)
            ).astype(jnp.float32)
    ch = jnp.dot(oh_s, ce_ref[:, :], preferred_element_type=jnp.float32)

    ratio = ratio_ref[0, 0]
    hw = jax.lax.div(r0, s)
    hpos = jax.lax.div(hw, w).astype(jnp.float32) * ratio
    wpos = jax.lax.rem(hw, w).astype(jnp.float32) * ratio
    k = jax.lax.broadcasted_iota(jnp.int32, (1, half), 1).astype(jnp.float32)
    omega = jnp.exp(k * (-math.log(10000.0) / half))
    ah = hpos * omega
    aw = wpos * omega
    a_r = jnp.concatenate([
        ch, jnp.zeros((r_blk, 2 * d4), dtype=jnp.float32),
        jnp.sin(ah), jnp.cos(ah), jnp.sin(aw), jnp.cos(aw),
    ], axis=-1)

    out_ref[...] = (tok_ref[...]
                    + a_t[None, None, :, :]
                    + a_r[None, :, None, :])


def kernel(tokens, timestamps, patch_size, input_res, channel_embed,
           pos_embed, month_table):
    b, h, w, t, s, d = tokens.shape
    d4 = d // 4
    r_blk = _R_BLK
    rows = h * w * s
    tok4 = tokens.transpose(0, 1, 2, 4, 3, 5).reshape(b, rows, t, d)
    months = timestamps[:, :, 1].astype(jnp.int32).reshape(b, t, 1)
    ratio = (jnp.float32(input_res) * jnp.float32(patch_size) / 10.0
             ).reshape(1, 1)

    grid = (b, rows // r_blk)
    out = pl.pallas_call(
        functools.partial(_body, t=t, s=s, w=w, d4=d4, r_blk=r_blk),
        grid=grid,
        in_specs=[
            pl.BlockSpec((1, t, 1), lambda i, j: (i, 0, 0)),
            pl.BlockSpec(channel_embed.shape, lambda i, j: (0, 0)),
            pl.BlockSpec(pos_embed.shape, lambda i, j: (0, 0)),
            pl.BlockSpec(month_table.shape, lambda i, j: (0, 0)),
            pl.BlockSpec(memory_space=pltpu.SMEM),
            pl.BlockSpec((1, r_blk, t, d), lambda i, j: (i, j, 0, 0)),
        ],
        out_specs=pl.BlockSpec((1, r_blk, t, d), lambda i, j: (i, j, 0, 0)),
        out_shape=jax.ShapeDtypeStruct((b, rows, t, d), jnp.float32),
        compiler_params=pltpu.CompilerParams(
            dimension_semantics=("arbitrary", "arbitrary")),
    )(months, channel_embed, pos_embed, month_table, ratio, tok4)
    return out.reshape(b, h, w, s, t, d).transpose(0, 1, 2, 4, 3, 5)


# speedup vs baseline: 1.0089x; 1.0089x over previous
import functools
import math

import jax
import jax.numpy as jnp
from jax.experimental import pallas as pl
from jax.experimental.pallas import tpu as pltpu

_R_BLK = 384


def _body(months_ref, ce_ref, pe_ref, mt_ref, ratio_ref, tok_ref, out_ref,
          *, t, s, w, d4, r_blk):
    half = d4 // 4

    pe = pe_ref[:t, :]
    m = months_ref[0]
    oh = (m == jax.lax.broadcasted_iota(jnp.int32, (t, 12), 1)).astype(jnp.float32)
    me = jnp.dot(oh, mt_ref[:, :], preferred_element_type=jnp.float32)
    zt = jnp.zeros((t, d4), dtype=jnp.float32)
    a_t = jnp.concatenate([zt, pe, me, zt], axis=-1)

    r0 = (pl.program_id(1) * r_blk
          + jax.lax.broadcasted_iota(jnp.int32, (r_blk, 1), 0))
    s_idx = jax.lax.rem(r0, s)
    oh_s = (s_idx == jax.lax.broadcasted_iota(jnp.int32, (r_blk, s), 1)
            ).astype(jnp.float32)
    ch = jnp.dot(oh_s, ce_ref[:, :], preferred_element_type=jnp.float32)

    ratio = ratio_ref[0, 0]
    hw = jax.lax.div(r0, s)
    hpos = jax.lax.div(hw, w).astype(jnp.float32) * ratio
    wpos = jax.lax.rem(hw, w).astype(jnp.float32) * ratio
    k = jax.lax.broadcasted_iota(jnp.int32, (1, half), 1).astype(jnp.float32)
    omega = jnp.exp(k * (-math.log(10000.0) / half))
    ah = hpos * omega
    aw = wpos * omega
    a_r = jnp.concatenate([
        ch, jnp.zeros((r_blk, 2 * d4), dtype=jnp.float32),
        jnp.sin(ah), jnp.cos(ah), jnp.sin(aw), jnp.cos(aw),
    ], axis=-1)

    out_ref[...] = (tok_ref[...]
                    + a_t[None, None, :, :]
                    + a_r[None, :, None, :])


def kernel(tokens, timestamps, patch_size, input_res, channel_embed,
           pos_embed, month_table):
    b, h, w, t, s, d = tokens.shape
    d4 = d // 4
    r_blk = _R_BLK
    rows = h * w * s
    tok4 = tokens.transpose(0, 1, 2, 4, 3, 5).reshape(b, rows, t, d)
    months = timestamps[:, :, 1].astype(jnp.int32).reshape(b, t, 1)
    ratio = (jnp.float32(input_res) * jnp.float32(patch_size) / 10.0
             ).reshape(1, 1)

    grid = (b, rows // r_blk)
    out = pl.pallas_call(
        functools.partial(_body, t=t, s=s, w=w, d4=d4, r_blk=r_blk),
        grid=grid,
        in_specs=[
            pl.BlockSpec((1, t, 1), lambda i, j: (i, 0, 0)),
            pl.BlockSpec(channel_embed.shape, lambda i, j: (0, 0)),
            pl.BlockSpec(pos_embed.shape, lambda i, j: (0, 0)),
            pl.BlockSpec(month_table.shape, lambda i, j: (0, 0)),
            pl.BlockSpec(memory_space=pltpu.SMEM),
            pl.BlockSpec((1, r_blk, t, d), lambda i, j: (i, j, 0, 0)),
        ],
        out_specs=pl.BlockSpec((1, r_blk, t, d), lambda i, j: (i, j, 0, 0)),
        out_shape=jax.ShapeDtypeStruct((b, rows, t, d), jnp.float32),
        compiler_params=pltpu.CompilerParams(
            dimension_semantics=("parallel", "parallel")),
    )(months, channel_embed, pos_embed, month_table, ratio, tok4)
    return out.reshape(b, h, w, s, t, d).transpose(0, 1, 2, 4, 3, 5)
